# baseline (speedup 1.0000x reference)
import jax
import jax.numpy as jnp
from jax import lax
from jax.experimental import pallas as pl
from jax.experimental.pallas import tpu as pltpu
from jax.experimental.pallas import tpu_sc as plsc

B, S = 1024, 200
BS = B * S
INTD = 32
HD = 64
H2 = HD // 2
EPS = 1e-6

NC, NS, L = 2, 16, 16
NW = NC * NS
TOK_W = BS // NW
CH = 128
NCH = TOK_W // CH
NROW = BS // CH

N_TEST, N_TAG, N_INTER, N_BIG = 1539, 914, 3, 10
OFF_TAG = 1544
OFF_INTER = 2464
OFF_BIG = 2472
SMALL_ROWS = 2488
NQ = 9456


def _proj_body(t0, t1, t2, t3, t4, t5, w_ref, pq_ref, pn_ref, ps_ref):
    w = w_ref[...]
    f32 = jnp.float32
    pq_ref[...] = jnp.dot(t1[...], w[1], preferred_element_type=f32)
    pn_ref[...] = jnp.dot(t4[...], w[4], preferred_element_type=f32)
    ps_ref[pl.ds(0, N_TEST)] = jnp.dot(t0[...], w[0],
                                       preferred_element_type=f32)
    ps_ref[pl.ds(OFF_TAG, N_TAG)] = jnp.dot(t2[...], w[2],
                                            preferred_element_type=f32)
    ps_ref[pl.ds(OFF_INTER, N_INTER)] = jnp.dot(t3[...], w[3],
                                                preferred_element_type=f32)
    ps_ref[pl.ds(OFF_BIG, N_BIG)] = jnp.dot(t5[...], w[5],
                                            preferred_element_type=f32)


def _project_tables(tables, w3):
    return pl.pallas_call(
        _proj_body,
        out_shape=(
            jax.ShapeDtypeStruct((NQ, H2), jnp.float32),
            jax.ShapeDtypeStruct((NQ, H2), jnp.float32),
            jax.ShapeDtypeStruct((SMALL_ROWS, H2), jnp.float32),
        ),
    )(*tables, w3)


def _sc_body(pq, pn, psmall, wcont, i0, i1, i2, i3, i4, i5, c1, c2, c3,
             out_hbm,
             x0, x1, cb0, cb1, rq0, rq1, rn0, rn1, a0, a1, small_v, wc_v,
             si0, si1, sg0, sg1, so0, so1):
    idxh = (i0, i1, i2, i3, i4, i5)
    conth = (c1, c2, c3)
    idxb = (x0, x1)
    cbb = (cb0, cb1)
    rqb = (rq0, rq1)
    rnb = (rn0, rn1)
    accb = (a0, a1)
    sib = (si0, si1)
    sgb = (sg0, sg1)
    sob = (so0, so1)
    wid = lax.axis_index("s") * NC + lax.axis_index("c")
    row0 = wid * NCH
    tok0 = wid * TOK_W

    def fire_idx(c, b):
        for f in range(6):
            pltpu.async_copy(idxh[f].at[row0 + c], idxb[b].at[f], sib[b])
        for k in range(3):
            pltpu.async_copy(conth[k].at[row0 + c], cbb[b].at[k], sib[b])

    def wait_idx(b):
        pltpu.make_async_copy(idxh[0].at[pl.ds(0, 6)], idxb[b], sib[b]).wait()
        pltpu.make_async_copy(conth[0].at[pl.ds(0, 3)], cbb[b], sib[b]).wait()

    def fire_g(b):
        pltpu.async_copy(pq.at[idxb[b].at[1]], rqb[b], sgb[b])
        pltpu.async_copy(pn.at[idxb[b].at[4]], rnb[b], sgb[b])

    def wait_g(b):
        pltpu.make_async_copy(pq.at[pl.ds(0, CH)], rqb[b], sgb[b]).wait()
        pltpu.make_async_copy(pn.at[pl.ds(0, CH)], rnb[b], sgb[b]).wait()

    def fire_out(c, b):
        pltpu.async_copy(accb[b], out_hbm.at[pl.ds(tok0 + c * CH, CH)], sob[b])

    def wait_out(b):
        pltpu.make_async_copy(
            accb[b], out_hbm.at[pl.ds(0, CH)], sob[b]).wait()

    def sum_chunk(b):
        iv, cv, rq, rn, acc = idxb[b], cbb[b], rqb[b], rnb[b], accb[b]

        def grp_body(g, car):
            t0 = g * L
            jv = [iv[0, pl.ds(t0, L)],
                  iv[2, pl.ds(t0, L)] + OFF_TAG,
                  iv[3, pl.ds(t0, L)] + OFF_INTER,
                  iv[5, pl.ds(t0, L)] + OFF_BIG]
            sv = [cv[k, pl.ds(t0, L)] for k in range(3)]
            wch = [[wc_v[k, pl.ds(h, L)] for k in range(3)] for h in (0, L)]
            for u in range(L):
                t = t0 + u
                js = [v[u] for v in jv]
                ss = [v[u] for v in sv]
                for hi, h in enumerate((0, L)):
                    a = rq[t, pl.ds(h, L)] + rn[t, pl.ds(h, L)]
                    for j in js:
                        a = a + small_v[j, pl.ds(h, L)]
                    acc[t, pl.ds(h, L)] = a
                    w0, w1, w2 = wch[hi]
                    acc[t, pl.ds(H2 + h, L)] = ss[0] * w0 + ss[1] * w1 + ss[2] * w2
            return car

        lax.fori_loop(0, CH // L, grp_body, 0)

    def step(c, b, first, fire2, nxt):
        if nxt:
            wait_idx(1 - b)
            fire_g(1 - b)
        wait_g(b)
        if not first:
            wait_out(b)
        sum_chunk(b)
        if fire2:
            fire_idx(c + 2, b)
        fire_out(c, b)

    pltpu.sync_copy(psmall, small_v)
    pltpu.sync_copy(wcont, wc_v)
    fire_idx(0, 0)
    fire_idx(1, 1)
    wait_idx(0)
    fire_g(0)
    step(0, 0, True, True, True)
    step(1, 1, True, True, True)

    def pair_body(k, car):
        c = 2 * k
        step(c, 0, False, True, True)
        step(c + 1, 1, False, True, True)
        return car

    lax.fori_loop(1, NCH // 2 - 1, pair_body, 0)
    step(NCH - 2, 0, False, False, True)
    step(NCH - 1, 1, False, False, False)
    wait_out(0)
    wait_out(1)


def _gather_sum(pq, pn, psmall, wcont, idxs, conts):
    mesh = plsc.VectorSubcoreMesh(
        core_axis_name="c", subcore_axis_name="s",
        num_cores=NC, num_subcores=NS,
    )
    scratch = (
        [pltpu.VMEM((6, CH), jnp.int32) for _ in range(2)]
        + [pltpu.VMEM((3, CH), jnp.float32) for _ in range(2)]
        + [pltpu.VMEM((CH, H2), jnp.float32) for _ in range(4)]
        + [pltpu.VMEM((CH, HD), jnp.float32) for _ in range(2)]
        + [pltpu.VMEM((SMALL_ROWS, H2), jnp.float32)]
        + [pltpu.VMEM((3, H2), jnp.float32)]
        + [pltpu.SemaphoreType.DMA for _ in range(6)]
    )
    kern = pl.kernel(
        _sc_body,
        out_type=jax.ShapeDtypeStruct((BS, HD), jnp.float32),
        mesh=mesh,
        scratch_types=scratch,
        compiler_params=pltpu.CompilerParams(use_tc_tiling_on_sc=False),
    )
    return kern(pq, pn, psmall, wcont, *idxs, *conts)


def _final_body(x_ref, b_ref, g_ref, bt_ref, out_ref):
    f32 = jnp.float32
    x = x_ref[...] + b_ref[...]
    rr = lax.broadcasted_iota(jnp.int32, (2 * HD, 2 * HD), 0) // H2
    cc = lax.broadcasted_iota(jnp.int32, (2 * HD, 2 * HD), 1) // H2
    mavg = jnp.where(rr == cc, 1.0 / H2, 0.0).astype(f32)
    m = jnp.dot(x, mavg, preferred_element_type=f32)
    xc = x - m
    v = jnp.dot(xc * xc, mavg, preferred_element_type=f32)
    out_ref[...] = xc * lax.rsqrt(v + EPS) * g_ref[...] + bt_ref[...]


def _finalize(x2, b128, g128, bt128):
    tb = 4096
    vec = lambda: pl.BlockSpec((2 * HD,), lambda i: (0,))
    return pl.pallas_call(
        _final_body,
        grid=(BS // 2 // tb,),
        in_specs=[
            pl.BlockSpec((tb, 2 * HD), lambda i: (i, 0)),
            vec(), vec(), vec(),
        ],
        out_specs=pl.BlockSpec((tb, 2 * HD), lambda i: (i, 0)),
        out_shape=jax.ShapeDtypeStruct((BS // 2, 2 * HD), jnp.float32),
    )(x2, b128, g128, bt128)


def kernel(testId, assessmentItemID, KnowledgeTag, interaction, question_N,
           bigclass, cont1, cont2, cont3,
           emb_test, emb_question, emb_tag, emb_interaction, emb_question_N,
           emb_bigclass,
           W_comb, b_comb, g_comb, beta_comb,
           W_cont, b_cont, g_cont, beta_cont):
    w3 = W_comb.reshape(6, INTD, H2)
    pq, pn, psmall = _project_tables(
        [emb_test, emb_question, emb_tag, emb_interaction, emb_question_N,
         emb_bigclass], w3)

    r = lambda a: a.reshape(NROW, CH).astype(jnp.int32)
    idxs = [r(testId), r(assessmentItemID), r(KnowledgeTag), r(interaction),
            r(question_N), r(bigclass)]
    rf = lambda a: a.reshape(NROW, CH)
    conts = [rf(cont1), rf(cont2), rf(cont3)]

    xpre = _gather_sum(pq, pn, psmall, W_cont, idxs, conts)
    b128 = jnp.concatenate([b_comb, b_cont, b_comb, b_cont])
    g128 = jnp.concatenate([g_comb, g_cont, g_comb, g_cont])
    bt128 = jnp.concatenate([beta_comb, beta_cont, beta_comb, beta_cont])
    out = _finalize(xpre.reshape(BS // 2, 2 * HD), b128, g128, bt128)
    return out.reshape(B, S, HD)
